# baseline (device time: 130787 ns/iter reference)
import jax
import jax.numpy as jnp
from jax import lax
from jax.experimental import pallas as pl
from jax.experimental.pallas import tpu as pltpu

N_Z = 4
H = 8
D = 128
SCALE = D ** -0.5
KV_CHUNK = 1024


def _partial_body(q_ref, k_hbm, v_hbm, u_ref, l_ref, kbuf, vbuf, ksems, vsems):
    b = u_ref.shape[0]
    skv = k_hbm.shape[1]
    nc = skv // KV_CHUNK
    nit = b * H * nc

    def params(i):
        return i // (H * nc), (i // nc) % H, i % nc

    def copies(i, slot):
        bb, hh, cc = params(i)
        ck = pltpu.make_async_copy(
            k_hbm.at[bb, pl.ds(cc * KV_CHUNK, KV_CHUNK), hh, :],
            kbuf.at[slot],
            ksems.at[slot],
        )
        cv = pltpu.make_async_copy(
            v_hbm.at[bb, pl.ds(cc * KV_CHUNK, KV_CHUNK), hh, :],
            vbuf.at[slot],
            vsems.at[slot],
        )
        return ck, cv

    l_ref[...] = jnp.zeros_like(l_ref)

    ck, cv = copies(0, 0)
    ck.start()
    cv.start()
    u_acc = None
    l_acc = None
    for i in range(nit):
        slot = i % 2
        if i + 1 < nit:
            ck2, cv2 = copies(i + 1, (i + 1) % 2)
            ck2.start()
            cv2.start()
        ck, cv = copies(i, slot)
        ck.wait()
        cv.wait()
        bb, hh, cc = params(i)
        q = q_ref[bb, :, hh, :]
        k = kbuf[slot]
        v = vbuf[slot]
        s = lax.dot_general(
            q, k, (((1,), (1,)), ((), ())), preferred_element_type=jnp.float32
        )
        p = jnp.exp(s * SCALE)
        lp = jnp.sum(p, axis=1, keepdims=True)
        up = lax.dot_general(
            p, v, (((1,), (0,)), ((), ())), preferred_element_type=jnp.float32
        )
        if cc == 0:
            u_acc, l_acc = up, lp
        else:
            u_acc, l_acc = u_acc + up, l_acc + lp
        if cc == nc - 1:
            u_ref[bb, :, hh, :] = u_acc
            l_ref[bb, :, hh:hh + 1] = l_acc


def _allreduce_body(
    u_ref, l_ref, o_ref, accl_ref, commu_ref, comml_ref,
    usend_sems, urecv_sems, lsend_sems, lrecv_sems,
):
    my_x = lax.axis_index("x")
    my_y = lax.axis_index("y")
    my_z = lax.axis_index("z")
    left = (my_z - 1) % N_Z
    right = (my_z + 1) % N_Z

    barrier_sem = pltpu.get_barrier_semaphore()
    for nbr in (left, right):
        pl.semaphore_signal(
            barrier_sem,
            inc=1,
            device_id=(my_x, my_y, nbr),
            device_id_type=pl.DeviceIdType.MESH,
        )
    pl.semaphore_wait(barrier_sem, 2)

    o_ref[...] = u_ref[...]
    accl_ref[...] = l_ref[...]
    commu_ref[0] = u_ref[...]
    comml_ref[0] = l_ref[...]

    for h in range(N_Z - 1):
        rdma_u = pltpu.make_async_remote_copy(
            src_ref=commu_ref.at[h],
            dst_ref=commu_ref.at[h + 1],
            send_sem=usend_sems.at[h],
            recv_sem=urecv_sems.at[h + 1],
            device_id=(my_x, my_y, right),
            device_id_type=pl.DeviceIdType.MESH,
        )
        rdma_l = pltpu.make_async_remote_copy(
            src_ref=comml_ref.at[h],
            dst_ref=comml_ref.at[h + 1],
            send_sem=lsend_sems.at[h],
            recv_sem=lrecv_sems.at[h + 1],
            device_id=(my_x, my_y, right),
            device_id_type=pl.DeviceIdType.MESH,
        )
        rdma_u.start()
        rdma_l.start()
        rdma_u.wait()
        rdma_l.wait()
        o_ref[...] += commu_ref[h + 1]
        accl_ref[...] += comml_ref[h + 1]

    for bb in range(o_ref.shape[0]):
        for hh in range(H):
            lcol = accl_ref[bb, :, hh:hh + 1]
            o_ref[bb, :, hh, :] = o_ref[bb, :, hh, :] / lcol


def partial_only(Q, K, V):
    b, sq, h, d = Q.shape

    return pl.pallas_call(
        _partial_body,
        in_specs=[
            pl.BlockSpec(memory_space=pltpu.VMEM),
            pl.BlockSpec(memory_space=pltpu.MemorySpace.HBM),
            pl.BlockSpec(memory_space=pltpu.MemorySpace.HBM),
        ],
        out_specs=[
            pl.BlockSpec(memory_space=pltpu.VMEM),
            pl.BlockSpec(memory_space=pltpu.VMEM),
        ],
        out_shape=[
            jax.ShapeDtypeStruct((b, sq, h, d), jnp.float32),
            jax.ShapeDtypeStruct((b, sq, d), jnp.float32),
        ],
        scratch_shapes=[
            pltpu.VMEM((2, KV_CHUNK, d), jnp.float32),
            pltpu.VMEM((2, KV_CHUNK, d), jnp.float32),
            pltpu.SemaphoreType.DMA((2,)),
            pltpu.SemaphoreType.DMA((2,)),
        ],
    )(Q, K, V)


def kernel(Q, K, V):
    b, sq, h, d = Q.shape
    u, l = partial_only(Q, K, V)

    return pl.pallas_call(
        _allreduce_body,
        in_specs=[
            pl.BlockSpec(memory_space=pltpu.VMEM),
            pl.BlockSpec(memory_space=pltpu.VMEM),
        ],
        out_specs=pl.BlockSpec(memory_space=pltpu.VMEM),
        out_shape=jax.ShapeDtypeStruct((b, sq, h, d), jnp.float32),
        scratch_shapes=[
            pltpu.VMEM((b, sq, d), jnp.float32),
            pltpu.VMEM((N_Z, b, sq, h, d), jnp.float32),
            pltpu.VMEM((N_Z, b, sq, d), jnp.float32),
            pltpu.SemaphoreType.DMA((N_Z,)),
            pltpu.SemaphoreType.DMA((N_Z,)),
            pltpu.SemaphoreType.DMA((N_Z,)),
            pltpu.SemaphoreType.DMA((N_Z,)),
        ],
        compiler_params=pltpu.CompilerParams(collective_id=0),
    )(u, l)


# device time: 84652 ns/iter; 1.5450x vs baseline; 1.5450x over previous
import jax
import jax.numpy as jnp
from jax import lax
from jax.experimental import pallas as pl
from jax.experimental.pallas import tpu as pltpu

N_Z = 4
H = 8
D = 128
SCALE = D ** -0.5
KV_CHUNK = 1024


def _partial_body(q_ref, k_hbm, v_hbm, u_ref, l_ref, kbuf, vbuf, ksems, vsems):
    b = u_ref.shape[0]
    skv = k_hbm.shape[1]
    nc = skv // KV_CHUNK
    nit = b * H * nc

    def params(i):
        return i // (H * nc), (i // nc) % H, i % nc

    def copies(i, slot):
        bb, hh, cc = params(i)
        ck = pltpu.make_async_copy(
            k_hbm.at[bb, pl.ds(cc * KV_CHUNK, KV_CHUNK), hh, :],
            kbuf.at[slot],
            ksems.at[slot],
        )
        cv = pltpu.make_async_copy(
            v_hbm.at[bb, pl.ds(cc * KV_CHUNK, KV_CHUNK), hh, :],
            vbuf.at[slot],
            vsems.at[slot],
        )
        return ck, cv

    l_ref[...] = jnp.zeros_like(l_ref)

    NBUF = kbuf.shape[0]
    for j in range(min(NBUF - 1, nit)):
        ckj, cvj = copies(j, j % NBUF)
        ckj.start()
        cvj.start()
    u_acc = None
    l_acc = None
    for i in range(nit):
        slot = i % NBUF
        if i + NBUF - 1 < nit:
            ck2, cv2 = copies(i + NBUF - 1, (i + NBUF - 1) % NBUF)
            ck2.start()
            cv2.start()
        ck, cv = copies(i, slot)
        ck.wait()
        cv.wait()
        bb, hh, cc = params(i)
        q = q_ref[bb, :, hh, :]
        k = kbuf[slot]
        v = vbuf[slot]
        s = lax.dot_general(
            q, k, (((1,), (1,)), ((), ())), preferred_element_type=jnp.float32
        )
        p = jnp.exp(s * SCALE)
        lp = jnp.sum(p, axis=1, keepdims=True)
        up = lax.dot_general(
            p, v, (((1,), (0,)), ((), ())), preferred_element_type=jnp.float32
        )
        if cc == 0:
            u_acc, l_acc = up, lp
        else:
            u_acc, l_acc = u_acc + up, l_acc + lp
        if cc == nc - 1:
            u_ref[bb, :, hh, :] = u_acc
            l_ref[bb, :, hh:hh + 1] = l_acc


def _allreduce_body(
    u_ref, l_ref, o_ref, accl_ref,
    lbu, rbu, slu, sru, lbl, rbl, sll, srl,
    lus, lur, rus, rur, lls, llr, rls, rlr,
):
    my_x = lax.axis_index("x")
    my_y = lax.axis_index("y")
    my_z = lax.axis_index("z")
    has_left = my_z > 0
    has_right = my_z < N_Z - 1
    left = jnp.maximum(my_z - 1, 0)
    right = jnp.minimum(my_z + 1, N_Z - 1)
    nb = u_ref.shape[0]

    barrier_sem = pltpu.get_barrier_semaphore()
    for nbr in ((my_z - 1) % N_Z, (my_z + 1) % N_Z):
        pl.semaphore_signal(
            barrier_sem,
            inc=1,
            device_id=(my_x, my_y, nbr),
            device_id_type=pl.DeviceIdType.MESH,
        )
    pl.semaphore_wait(barrier_sem, 2)

    def mk(src, dst, ssem, rsem, dz):
        return pltpu.make_async_remote_copy(
            src_ref=src,
            dst_ref=dst,
            send_sem=ssem,
            recv_sem=rsem,
            device_id=(my_x, my_y, dz),
            device_id_type=pl.DeviceIdType.MESH,
        )

    for c in range(nb):
        @pl.when(jnp.logical_and(has_right, has_left))
        def _(c=c):
            mk(lbu.at[c], lbu.at[c], lus.at[c], lur.at[c], left).wait_recv()
            mk(lbl.at[c], lbl.at[c], lls.at[c], llr.at[c], left).wait_recv()

        @pl.when(has_right)
        def _(c=c):
            slu[c] = u_ref[c] + jnp.where(has_left, lbu[c], 0.0)
            sll[c] = l_ref[c] + jnp.where(has_left, lbl[c], 0.0)
            mk(slu.at[c], lbu.at[c], lus.at[c], lur.at[c], right).start()
            mk(sll.at[c], lbl.at[c], lls.at[c], llr.at[c], right).start()

        @pl.when(jnp.logical_and(has_left, has_right))
        def _(c=c):
            mk(rbu.at[c], rbu.at[c], rus.at[c], rur.at[c], right).wait_recv()
            mk(rbl.at[c], rbl.at[c], rls.at[c], rlr.at[c], right).wait_recv()

        @pl.when(has_left)
        def _(c=c):
            sru[c] = u_ref[c] + jnp.where(has_right, rbu[c], 0.0)
            srl[c] = l_ref[c] + jnp.where(has_right, rbl[c], 0.0)
            mk(sru.at[c], rbu.at[c], rus.at[c], rur.at[c], left).start()
            mk(srl.at[c], rbl.at[c], rls.at[c], rlr.at[c], left).start()

    for c in range(nb):
        @pl.when(jnp.logical_and(has_left, jnp.logical_not(has_right)))
        def _(c=c):
            mk(lbu.at[c], lbu.at[c], lus.at[c], lur.at[c], left).wait_recv()
            mk(lbl.at[c], lbl.at[c], lls.at[c], llr.at[c], left).wait_recv()

        @pl.when(jnp.logical_and(has_right, jnp.logical_not(has_left)))
        def _(c=c):
            mk(rbu.at[c], rbu.at[c], rus.at[c], rur.at[c], right).wait_recv()
            mk(rbl.at[c], rbl.at[c], rls.at[c], rlr.at[c], right).wait_recv()

        o_ref[c] = (
            u_ref[c]
            + jnp.where(has_left, lbu[c], 0.0)
            + jnp.where(has_right, rbu[c], 0.0)
        )
        accl_ref[c] = (
            l_ref[c]
            + jnp.where(has_left, lbl[c], 0.0)
            + jnp.where(has_right, rbl[c], 0.0)
        )

    for c in range(nb):
        @pl.when(has_right)
        def _(c=c):
            mk(slu.at[c], lbu.at[c], lus.at[c], lur.at[c], right).wait_send()
            mk(sll.at[c], lbl.at[c], lls.at[c], llr.at[c], right).wait_send()

        @pl.when(has_left)
        def _(c=c):
            mk(sru.at[c], rbu.at[c], rus.at[c], rur.at[c], left).wait_send()
            mk(srl.at[c], rbl.at[c], rls.at[c], rlr.at[c], left).wait_send()

    for bb in range(o_ref.shape[0]):
        for hh in range(H):
            lcol = accl_ref[bb, :, hh:hh + 1]
            o_ref[bb, :, hh, :] = o_ref[bb, :, hh, :] / lcol


def partial_only(Q, K, V):
    b, sq, h, d = Q.shape

    return pl.pallas_call(
        _partial_body,
        in_specs=[
            pl.BlockSpec(memory_space=pltpu.VMEM),
            pl.BlockSpec(memory_space=pltpu.MemorySpace.HBM),
            pl.BlockSpec(memory_space=pltpu.MemorySpace.HBM),
        ],
        out_specs=[
            pl.BlockSpec(memory_space=pltpu.VMEM),
            pl.BlockSpec(memory_space=pltpu.VMEM),
        ],
        out_shape=[
            jax.ShapeDtypeStruct((b, sq, h, d), jnp.float32),
            jax.ShapeDtypeStruct((b, sq, d), jnp.float32),
        ],
        scratch_shapes=[
            pltpu.VMEM((8, KV_CHUNK, d), jnp.float32),
            pltpu.VMEM((8, KV_CHUNK, d), jnp.float32),
            pltpu.SemaphoreType.DMA((8,)),
            pltpu.SemaphoreType.DMA((8,)),
        ],
    )(Q, K, V)


def kernel(Q, K, V):
    b, sq, h, d = Q.shape
    u, l = partial_only(Q, K, V)

    return pl.pallas_call(
        _allreduce_body,
        in_specs=[
            pl.BlockSpec(memory_space=pltpu.VMEM),
            pl.BlockSpec(memory_space=pltpu.VMEM),
        ],
        out_specs=pl.BlockSpec(memory_space=pltpu.VMEM),
        out_shape=jax.ShapeDtypeStruct((b, sq, h, d), jnp.float32),
        scratch_shapes=[
            pltpu.VMEM((b, sq, d), jnp.float32),
            pltpu.VMEM((b, sq, h, d), jnp.float32),
            pltpu.VMEM((b, sq, h, d), jnp.float32),
            pltpu.VMEM((b, sq, h, d), jnp.float32),
            pltpu.VMEM((b, sq, h, d), jnp.float32),
            pltpu.VMEM((b, sq, d), jnp.float32),
            pltpu.VMEM((b, sq, d), jnp.float32),
            pltpu.VMEM((b, sq, d), jnp.float32),
            pltpu.VMEM((b, sq, d), jnp.float32),
            pltpu.SemaphoreType.DMA((b,)),
            pltpu.SemaphoreType.DMA((b,)),
            pltpu.SemaphoreType.DMA((b,)),
            pltpu.SemaphoreType.DMA((b,)),
            pltpu.SemaphoreType.DMA((b,)),
            pltpu.SemaphoreType.DMA((b,)),
            pltpu.SemaphoreType.DMA((b,)),
            pltpu.SemaphoreType.DMA((b,)),
        ],
        compiler_params=pltpu.CompilerParams(collective_id=0),
    )(u, l)


# device time: 67031 ns/iter; 1.9511x vs baseline; 1.2629x over previous
import jax
import jax.numpy as jnp
from jax import lax
from jax.experimental import pallas as pl
from jax.experimental.pallas import tpu as pltpu

N_Z = 4
H = 8
D = 128
SCALE = D ** -0.5
KV_CHUNK = 1024


def _partial_body(q_ref, k_hbm, v_hbm, u_ref, l_ref, kbuf, vbuf, ksems, vsems):
    b = u_ref.shape[0]
    skv = k_hbm.shape[1]
    nc = skv // KV_CHUNK
    nit = b * H * nc

    def params(i):
        return i // (H * nc), (i // nc) % H, i % nc

    def copies(i, slot):
        bb, hh, cc = params(i)
        ck = pltpu.make_async_copy(
            k_hbm.at[bb, pl.ds(cc * KV_CHUNK, KV_CHUNK), hh, :],
            kbuf.at[slot],
            ksems.at[slot],
        )
        cv = pltpu.make_async_copy(
            v_hbm.at[bb, pl.ds(cc * KV_CHUNK, KV_CHUNK), hh, :],
            vbuf.at[slot],
            vsems.at[slot],
        )
        return ck, cv

    l_ref[...] = jnp.zeros_like(l_ref)

    NBUF = kbuf.shape[0]
    for j in range(min(NBUF - 1, nit)):
        ckj, cvj = copies(j, j % NBUF)
        ckj.start()
        cvj.start()
    u_acc = None
    l_acc = None
    for i in range(nit):
        slot = i % NBUF
        if i + NBUF - 1 < nit:
            ck2, cv2 = copies(i + NBUF - 1, (i + NBUF - 1) % NBUF)
            ck2.start()
            cv2.start()
        ck, cv = copies(i, slot)
        ck.wait()
        cv.wait()
        bb, hh, cc = params(i)
        q = q_ref[bb, :, hh, :]
        k = kbuf[slot]
        v = vbuf[slot]
        s = lax.dot_general(
            q, k, (((1,), (1,)), ((), ())), preferred_element_type=jnp.float32
        )
        p = jnp.exp(s * SCALE)
        lp = jnp.sum(p, axis=1, keepdims=True)
        up = lax.dot_general(
            p, v, (((1,), (0,)), ((), ())), preferred_element_type=jnp.float32
        )
        if cc == 0:
            u_acc, l_acc = up, lp
        else:
            u_acc, l_acc = u_acc + up, l_acc + lp
        if cc == nc - 1:
            u_ref[bb, :, hh, :] = u_acc
            l_ref[bb, :, hh:hh + 1] = l_acc


def _allreduce_body(
    u_ref, l_ref, o_ref, accl_ref,
    zbu, zbl, zstu, zstl, xbu, xbl, ystu, ystl, ybu, ybl,
    zsend_u, zrecv_u, zsend_l, zrecv_l,
    xsu, xru, xsl, xrl, ysu, yru, ysl, yrl,
):
    my_x = lax.axis_index("x")
    my_y = lax.axis_index("y")
    my_z = lax.axis_index("z")
    q_idx = 2 * my_x + my_y

    def mk(src, dst, ssem, rsem, did):
        return pltpu.make_async_remote_copy(
            src_ref=src,
            dst_ref=dst,
            send_sem=ssem,
            recv_sem=rsem,
            device_id=did,
            device_id_type=pl.DeviceIdType.MESH,
        )

    barrier_sem = pltpu.get_barrier_semaphore()
    for zz in range(N_Z):
        @pl.when(my_z != zz)
        def _(zz=zz):
            pl.semaphore_signal(
                barrier_sem, inc=1,
                device_id=(my_x, my_y, zz),
                device_id_type=pl.DeviceIdType.MESH,
            )
    for did in ((1 - my_x, my_y, my_z), (my_x, 1 - my_y, my_z)):
        pl.semaphore_signal(
            barrier_sem, inc=1, device_id=did,
            device_id_type=pl.DeviceIdType.MESH,
        )
    pl.semaphore_wait(barrier_sem, N_Z - 1 + 2)

    for qq in range(N_Z):
        @pl.when(q_idx == qq)
        def _(qq=qq):
            zstu[...] = u_ref[qq:qq + 1]
            zstl[...] = l_ref[qq:qq + 1]

    for me in range(N_Z):
        @pl.when(my_z == me)
        def _(me=me):
            for zz in range(N_Z):
                if zz == me:
                    continue
                mk(zstu, zbu.at[me], zsend_u.at[zz], zrecv_u.at[me],
                   (my_x, my_y, zz)).start()
                mk(zstl, zbl.at[me], zsend_l.at[zz], zrecv_l.at[me],
                   (my_x, my_y, zz)).start()
            for zz in range(N_Z):
                if zz == me:
                    continue
                mk(zbu.at[zz], zbu.at[zz], zsend_u.at[zz], zrecv_u.at[zz],
                   (my_x, my_y, zz)).wait_recv()
                mk(zbl.at[zz], zbl.at[zz], zsend_l.at[zz], zrecv_l.at[zz],
                   (my_x, my_y, zz)).wait_recv()

    zsum_u = zstu[...]
    zsum_l = zstl[...]
    for zz in range(N_Z):
        zsum_u = zsum_u + jnp.where(my_z != zz, zbu[zz], 0.0)
        zsum_l = zsum_l + jnp.where(my_z != zz, zbl[zz], 0.0)
    zstu[...] = zsum_u
    zstl[...] = zsum_l

    x_did = (1 - my_x, my_y, my_z)
    mk(zstu, xbu, xsu, xru, x_did).start()
    mk(zstl, xbl, xsl, xrl, x_did).start()
    mk(xbu, xbu, xsu, xru, x_did).wait_recv()
    mk(xbl, xbl, xsl, xrl, x_did).wait_recv()

    ystu[0:1] = zstu[...]
    ystu[1:2] = xbu[...]
    ystl[0:1] = zstl[...]
    ystl[1:2] = xbl[...]
    y_did = (my_x, 1 - my_y, my_z)
    mk(ystu, ybu, ysu, yru, y_did).start()
    mk(ystl, ybl, ysl, yrl, y_did).start()
    mk(ybu, ybu, ysu, yru, y_did).wait_recv()
    mk(ybl, ybl, ysl, yrl, y_did).wait_recv()

    q_x = 2 * (1 - my_x) + my_y
    q_y0 = 2 * my_x + (1 - my_y)
    q_y1 = 2 * (1 - my_x) + (1 - my_y)
    for c in range(N_Z):
        o_ref[c:c + 1] = (
            jnp.where(q_idx == c, zstu[...], 0.0)
            + jnp.where(q_x == c, xbu[...], 0.0)
            + jnp.where(q_y0 == c, ybu[0:1], 0.0)
            + jnp.where(q_y1 == c, ybu[1:2], 0.0)
        )
        accl_ref[c:c + 1] = (
            jnp.where(q_idx == c, zstl[...], 0.0)
            + jnp.where(q_x == c, xbl[...], 0.0)
            + jnp.where(q_y0 == c, ybl[0:1], 0.0)
            + jnp.where(q_y1 == c, ybl[1:2], 0.0)
        )

    for me in range(N_Z):
        @pl.when(my_z == me)
        def _(me=me):
            for zz in range(N_Z):
                if zz == me:
                    continue
                mk(zstu, zbu.at[me], zsend_u.at[zz], zrecv_u.at[me],
                   (my_x, my_y, zz)).wait_send()
                mk(zstl, zbl.at[me], zsend_l.at[zz], zrecv_l.at[me],
                   (my_x, my_y, zz)).wait_send()
    mk(zstu, xbu, xsu, xru, x_did).wait_send()
    mk(zstl, xbl, xsl, xrl, x_did).wait_send()
    mk(ystu, ybu, ysu, yru, y_did).wait_send()
    mk(ystl, ybl, ysl, yrl, y_did).wait_send()

    for bb in range(o_ref.shape[0]):
        for hh in range(H):
            lcol = accl_ref[bb, :, hh:hh + 1]
            o_ref[bb, :, hh, :] = o_ref[bb, :, hh, :] / lcol


def partial_only(Q, K, V):
    b, sq, h, d = Q.shape

    return pl.pallas_call(
        _partial_body,
        in_specs=[
            pl.BlockSpec(memory_space=pltpu.VMEM),
            pl.BlockSpec(memory_space=pltpu.MemorySpace.HBM),
            pl.BlockSpec(memory_space=pltpu.MemorySpace.HBM),
        ],
        out_specs=[
            pl.BlockSpec(memory_space=pltpu.VMEM),
            pl.BlockSpec(memory_space=pltpu.VMEM),
        ],
        out_shape=[
            jax.ShapeDtypeStruct((b, sq, h, d), jnp.float32),
            jax.ShapeDtypeStruct((b, sq, d), jnp.float32),
        ],
        scratch_shapes=[
            pltpu.VMEM((8, KV_CHUNK, d), jnp.float32),
            pltpu.VMEM((8, KV_CHUNK, d), jnp.float32),
            pltpu.SemaphoreType.DMA((8,)),
            pltpu.SemaphoreType.DMA((8,)),
        ],
    )(Q, K, V)


def allreduce_only(u, l):
    b, sq, h, d = u.shape

    return pl.pallas_call(
        _allreduce_body,
        in_specs=[
            pl.BlockSpec(memory_space=pltpu.VMEM),
            pl.BlockSpec(memory_space=pltpu.VMEM),
        ],
        out_specs=pl.BlockSpec(memory_space=pltpu.VMEM),
        out_shape=jax.ShapeDtypeStruct((b, sq, h, d), jnp.float32),
        scratch_shapes=[
            pltpu.VMEM((b, sq, d), jnp.float32),
            pltpu.VMEM((N_Z, 1, sq, h, d), jnp.float32),
            pltpu.VMEM((N_Z, 1, sq, d), jnp.float32),
            pltpu.VMEM((1, sq, h, d), jnp.float32),
            pltpu.VMEM((1, sq, d), jnp.float32),
            pltpu.VMEM((1, sq, h, d), jnp.float32),
            pltpu.VMEM((1, sq, d), jnp.float32),
            pltpu.VMEM((2, sq, h, d), jnp.float32),
            pltpu.VMEM((2, sq, d), jnp.float32),
            pltpu.VMEM((2, sq, h, d), jnp.float32),
            pltpu.VMEM((2, sq, d), jnp.float32),
            pltpu.SemaphoreType.DMA((N_Z,)),
            pltpu.SemaphoreType.DMA((N_Z,)),
            pltpu.SemaphoreType.DMA((N_Z,)),
            pltpu.SemaphoreType.DMA((N_Z,)),
            pltpu.SemaphoreType.DMA,
            pltpu.SemaphoreType.DMA,
            pltpu.SemaphoreType.DMA,
            pltpu.SemaphoreType.DMA,
            pltpu.SemaphoreType.DMA,
            pltpu.SemaphoreType.DMA,
            pltpu.SemaphoreType.DMA,
            pltpu.SemaphoreType.DMA,
        ],
        compiler_params=pltpu.CompilerParams(collective_id=0),
    )(u, l)


def kernel(Q, K, V):
    u, l = partial_only(Q, K, V)
    return allreduce_only(u, l)


# device time: 35152 ns/iter; 3.7206x vs baseline; 1.9069x over previous
import jax
import jax.numpy as jnp
from jax import lax
from jax.experimental import pallas as pl
from jax.experimental.pallas import tpu as pltpu

N_Z = 4
H = 8
D = 128
SCALE = D ** -0.5
KV_CHUNK = 1024


def _partial_body(q_ref, k_hbm, v_hbm, u_ref, l_ref, kbuf, vbuf, ksems, vsems):
    skv = k_hbm.shape[1]
    nc = skv // KV_CHUNK
    nit = H * nc
    q_idx = 2 * lax.axis_index("x") + lax.axis_index("y")

    l_ref[...] = jnp.zeros_like(l_ref)

    for qq in range(N_Z):
        @pl.when(q_idx == qq)
        def _(qq=qq):
            def params(i):
                return i // nc, i % nc

            def copies(i, slot):
                hh, cc = params(i)
                ck = pltpu.make_async_copy(
                    k_hbm.at[qq, pl.ds(cc * KV_CHUNK, KV_CHUNK), hh, :],
                    kbuf.at[slot],
                    ksems.at[slot],
                )
                cv = pltpu.make_async_copy(
                    v_hbm.at[qq, pl.ds(cc * KV_CHUNK, KV_CHUNK), hh, :],
                    vbuf.at[slot],
                    vsems.at[slot],
                )
                return ck, cv

            NBUF = kbuf.shape[0]
            for j in range(min(NBUF - 1, nit)):
                ckj, cvj = copies(j, j % NBUF)
                ckj.start()
                cvj.start()
            u_acc = None
            l_acc = None
            for i in range(nit):
                slot = i % NBUF
                if i + NBUF - 1 < nit:
                    ck2, cv2 = copies(i + NBUF - 1, (i + NBUF - 1) % NBUF)
                    ck2.start()
                    cv2.start()
                ck, cv = copies(i, slot)
                ck.wait()
                cv.wait()
                hh, cc = params(i)
                q = q_ref[qq, :, hh, :]
                k = kbuf[slot]
                v = vbuf[slot]
                s = lax.dot_general(
                    q, k, (((1,), (1,)), ((), ())),
                    preferred_element_type=jnp.float32,
                )
                p = jnp.exp(s * SCALE)
                lp = jnp.sum(p, axis=1, keepdims=True)
                up = lax.dot_general(
                    p, v, (((1,), (0,)), ((), ())),
                    preferred_element_type=jnp.float32,
                )
                if cc == 0:
                    u_acc, l_acc = up, lp
                else:
                    u_acc, l_acc = u_acc + up, l_acc + lp
                if cc == nc - 1:
                    u_ref[0, :, hh, :] = u_acc
                    l_ref[0, :, hh:hh + 1] = l_acc


def _allreduce_body(
    u_ref, l_ref, o_ref, accl_ref,
    zbu, zbl, zstu, zstl, xbu, xbl, ystu, ystl, ybu, ybl,
    zsend_u, zrecv_u, zsend_l, zrecv_l,
    xsu, xru, xsl, xrl, ysu, yru, ysl, yrl,
):
    my_x = lax.axis_index("x")
    my_y = lax.axis_index("y")
    my_z = lax.axis_index("z")
    q_idx = 2 * my_x + my_y

    def mk(src, dst, ssem, rsem, did):
        return pltpu.make_async_remote_copy(
            src_ref=src,
            dst_ref=dst,
            send_sem=ssem,
            recv_sem=rsem,
            device_id=did,
            device_id_type=pl.DeviceIdType.MESH,
        )

    barrier_sem = pltpu.get_barrier_semaphore()
    for zz in range(N_Z):
        @pl.when(my_z != zz)
        def _(zz=zz):
            pl.semaphore_signal(
                barrier_sem, inc=1,
                device_id=(my_x, my_y, zz),
                device_id_type=pl.DeviceIdType.MESH,
            )
    for did in ((1 - my_x, my_y, my_z), (my_x, 1 - my_y, my_z)):
        pl.semaphore_signal(
            barrier_sem, inc=1, device_id=did,
            device_id_type=pl.DeviceIdType.MESH,
        )
    pl.semaphore_wait(barrier_sem, N_Z - 1 + 2)

    zstu[...] = u_ref[...]
    zstl[...] = l_ref[...]

    for me in range(N_Z):
        @pl.when(my_z == me)
        def _(me=me):
            for zz in range(N_Z):
                if zz == me:
                    continue
                mk(zstu, zbu.at[me], zsend_u.at[zz], zrecv_u.at[me],
                   (my_x, my_y, zz)).start()
                mk(zstl, zbl.at[me], zsend_l.at[zz], zrecv_l.at[me],
                   (my_x, my_y, zz)).start()
            for zz in range(N_Z):
                if zz == me:
                    continue
                mk(zbu.at[zz], zbu.at[zz], zsend_u.at[zz], zrecv_u.at[zz],
                   (my_x, my_y, zz)).wait_recv()
                mk(zbl.at[zz], zbl.at[zz], zsend_l.at[zz], zrecv_l.at[zz],
                   (my_x, my_y, zz)).wait_recv()

    zsum_u = zstu[...]
    zsum_l = zstl[...]
    for zz in range(N_Z):
        zsum_u = zsum_u + jnp.where(my_z != zz, zbu[zz], 0.0)
        zsum_l = zsum_l + jnp.where(my_z != zz, zbl[zz], 0.0)
    zstu[...] = zsum_u
    zstl[...] = zsum_l

    x_did = (1 - my_x, my_y, my_z)
    mk(zstu, xbu, xsu, xru, x_did).start()
    mk(zstl, xbl, xsl, xrl, x_did).start()
    mk(xbu, xbu, xsu, xru, x_did).wait_recv()
    mk(xbl, xbl, xsl, xrl, x_did).wait_recv()

    ystu[0:1] = zstu[...]
    ystu[1:2] = xbu[...]
    ystl[0:1] = zstl[...]
    ystl[1:2] = xbl[...]
    y_did = (my_x, 1 - my_y, my_z)
    mk(ystu, ybu, ysu, yru, y_did).start()
    mk(ystl, ybl, ysl, yrl, y_did).start()
    mk(ybu, ybu, ysu, yru, y_did).wait_recv()
    mk(ybl, ybl, ysl, yrl, y_did).wait_recv()

    q_x = 2 * (1 - my_x) + my_y
    q_y0 = 2 * my_x + (1 - my_y)
    q_y1 = 2 * (1 - my_x) + (1 - my_y)
    for c in range(N_Z):
        o_ref[c:c + 1] = (
            jnp.where(q_idx == c, zstu[...], 0.0)
            + jnp.where(q_x == c, xbu[...], 0.0)
            + jnp.where(q_y0 == c, ybu[0:1], 0.0)
            + jnp.where(q_y1 == c, ybu[1:2], 0.0)
        )
        accl_ref[c:c + 1] = (
            jnp.where(q_idx == c, zstl[...], 0.0)
            + jnp.where(q_x == c, xbl[...], 0.0)
            + jnp.where(q_y0 == c, ybl[0:1], 0.0)
            + jnp.where(q_y1 == c, ybl[1:2], 0.0)
        )

    for me in range(N_Z):
        @pl.when(my_z == me)
        def _(me=me):
            for zz in range(N_Z):
                if zz == me:
                    continue
                mk(zstu, zbu.at[me], zsend_u.at[zz], zrecv_u.at[me],
                   (my_x, my_y, zz)).wait_send()
                mk(zstl, zbl.at[me], zsend_l.at[zz], zrecv_l.at[me],
                   (my_x, my_y, zz)).wait_send()
    mk(zstu, xbu, xsu, xru, x_did).wait_send()
    mk(zstl, xbl, xsl, xrl, x_did).wait_send()
    mk(ystu, ybu, ysu, yru, y_did).wait_send()
    mk(ystl, ybl, ysl, yrl, y_did).wait_send()

    for bb in range(o_ref.shape[0]):
        for hh in range(H):
            lcol = accl_ref[bb, :, hh:hh + 1]
            o_ref[bb, :, hh, :] = o_ref[bb, :, hh, :] / lcol


def partial_only(Q, K, V):
    b, sq, h, d = Q.shape

    return pl.pallas_call(
        _partial_body,
        in_specs=[
            pl.BlockSpec(memory_space=pltpu.VMEM),
            pl.BlockSpec(memory_space=pltpu.MemorySpace.HBM),
            pl.BlockSpec(memory_space=pltpu.MemorySpace.HBM),
        ],
        out_specs=[
            pl.BlockSpec(memory_space=pltpu.VMEM),
            pl.BlockSpec(memory_space=pltpu.VMEM),
        ],
        out_shape=[
            jax.ShapeDtypeStruct((1, sq, h, d), jnp.float32),
            jax.ShapeDtypeStruct((1, sq, d), jnp.float32),
        ],
        scratch_shapes=[
            pltpu.VMEM((8, KV_CHUNK, d), jnp.float32),
            pltpu.VMEM((8, KV_CHUNK, d), jnp.float32),
            pltpu.SemaphoreType.DMA((8,)),
            pltpu.SemaphoreType.DMA((8,)),
        ],
    )(Q, K, V)


def allreduce_only(u, l):
    _, sq, h, d = u.shape
    b = N_Z

    return pl.pallas_call(
        _allreduce_body,
        in_specs=[
            pl.BlockSpec(memory_space=pltpu.VMEM),
            pl.BlockSpec(memory_space=pltpu.VMEM),
        ],
        out_specs=pl.BlockSpec(memory_space=pltpu.VMEM),
        out_shape=jax.ShapeDtypeStruct((b, sq, h, d), jnp.float32),
        scratch_shapes=[
            pltpu.VMEM((b, sq, d), jnp.float32),
            pltpu.VMEM((N_Z, 1, sq, h, d), jnp.float32),
            pltpu.VMEM((N_Z, 1, sq, d), jnp.float32),
            pltpu.VMEM((1, sq, h, d), jnp.float32),
            pltpu.VMEM((1, sq, d), jnp.float32),
            pltpu.VMEM((1, sq, h, d), jnp.float32),
            pltpu.VMEM((1, sq, d), jnp.float32),
            pltpu.VMEM((2, sq, h, d), jnp.float32),
            pltpu.VMEM((2, sq, d), jnp.float32),
            pltpu.VMEM((2, sq, h, d), jnp.float32),
            pltpu.VMEM((2, sq, d), jnp.float32),
            pltpu.SemaphoreType.DMA((N_Z,)),
            pltpu.SemaphoreType.DMA((N_Z,)),
            pltpu.SemaphoreType.DMA((N_Z,)),
            pltpu.SemaphoreType.DMA((N_Z,)),
            pltpu.SemaphoreType.DMA,
            pltpu.SemaphoreType.DMA,
            pltpu.SemaphoreType.DMA,
            pltpu.SemaphoreType.DMA,
            pltpu.SemaphoreType.DMA,
            pltpu.SemaphoreType.DMA,
            pltpu.SemaphoreType.DMA,
            pltpu.SemaphoreType.DMA,
        ],
        compiler_params=pltpu.CompilerParams(collective_id=0),
    )(u, l)


def kernel(Q, K, V):
    u, l = partial_only(Q, K, V)
    return allreduce_only(u, l)


# device time: 33595 ns/iter; 3.8930x vs baseline; 1.0463x over previous
import jax
import jax.numpy as jnp
from jax import lax
from jax.experimental import pallas as pl
from jax.experimental.pallas import tpu as pltpu

N_Z = 4
H = 8
D = 128
SCALE = D ** -0.5
KV_CHUNK = 1024


def _partial_body(q_ref, k_hbm, v_hbm, u_ref, l_ref, kbuf, vbuf, ksems, vsems):
    skv = k_hbm.shape[1]
    nc = skv // KV_CHUNK
    nit = H * nc
    q_idx = 2 * lax.axis_index("x") + lax.axis_index("y")

    l_ref[...] = jnp.zeros_like(l_ref)

    for qq in range(N_Z):
        @pl.when(q_idx == qq)
        def _(qq=qq):
            def params(i):
                return i // nc, i % nc

            def copies(i, slot):
                hh, cc = params(i)
                ck = pltpu.make_async_copy(
                    k_hbm.at[qq, pl.ds(cc * KV_CHUNK, KV_CHUNK), hh, :],
                    kbuf.at[slot],
                    ksems.at[slot],
                )
                cv = pltpu.make_async_copy(
                    v_hbm.at[qq, pl.ds(cc * KV_CHUNK, KV_CHUNK), hh, :],
                    vbuf.at[slot],
                    vsems.at[slot],
                )
                return ck, cv

            NBUF = kbuf.shape[0]
            for j in range(min(NBUF - 1, nit)):
                ckj, cvj = copies(j, j % NBUF)
                ckj.start()
                cvj.start()
            u_acc = None
            l_acc = None
            for i in range(nit):
                slot = i % NBUF
                if i + NBUF - 1 < nit:
                    ck2, cv2 = copies(i + NBUF - 1, (i + NBUF - 1) % NBUF)
                    ck2.start()
                    cv2.start()
                ck, cv = copies(i, slot)
                ck.wait()
                cv.wait()
                hh, cc = params(i)
                q = q_ref[qq, :, hh, :]
                k = kbuf[slot]
                v = vbuf[slot]
                s = lax.dot_general(
                    q, k, (((1,), (1,)), ((), ())),
                    preferred_element_type=jnp.float32,
                )
                p = jnp.exp(s * SCALE)
                lp = jnp.sum(p, axis=1, keepdims=True)
                up = lax.dot_general(
                    p, v, (((1,), (0,)), ((), ())),
                    preferred_element_type=jnp.float32,
                )
                if cc == 0:
                    u_acc, l_acc = up, lp
                else:
                    u_acc, l_acc = u_acc + up, l_acc + lp
                if cc == nc - 1:
                    u_ref[0, :, hh, :] = u_acc
                    l_ref[0, :, hh:hh + 1] = l_acc


def _allreduce_body(
    u_ref, l_ref, o_ref, accl_ref,
    zbu, zbl, zstu, zstl, xbu, xbl, ybu, ybl,
    zsend_u, zrecv_u, zsend_l, zrecv_l,
    xsu, xru, xsl, xrl, ysu, yru, ysl, yrl,
):
    my_x = lax.axis_index("x")
    my_y = lax.axis_index("y")
    my_z = lax.axis_index("z")
    q_idx = 2 * my_x + my_y

    def mk(src, dst, ssem, rsem, did):
        return pltpu.make_async_remote_copy(
            src_ref=src,
            dst_ref=dst,
            send_sem=ssem,
            recv_sem=rsem,
            device_id=did,
            device_id_type=pl.DeviceIdType.MESH,
        )

    barrier_sem = pltpu.get_barrier_semaphore()
    for zz in range(N_Z):
        @pl.when(my_z != zz)
        def _(zz=zz):
            pl.semaphore_signal(
                barrier_sem, inc=1,
                device_id=(my_x, my_y, zz),
                device_id_type=pl.DeviceIdType.MESH,
            )
    for did in ((1 - my_x, my_y, my_z), (my_x, 1 - my_y, my_z)):
        pl.semaphore_signal(
            barrier_sem, inc=1, device_id=did,
            device_id_type=pl.DeviceIdType.MESH,
        )
    pl.semaphore_wait(barrier_sem, N_Z - 1 + 2)

    zstu[...] = u_ref[...]
    zstl[...] = l_ref[...]

    for me in range(N_Z):
        @pl.when(my_z == me)
        def _(me=me):
            for zz in range(N_Z):
                if zz == me:
                    continue
                mk(zstu, zbu.at[me], zsend_u.at[zz], zrecv_u.at[me],
                   (my_x, my_y, zz)).start()
                mk(zstl, zbl.at[me], zsend_l.at[zz], zrecv_l.at[me],
                   (my_x, my_y, zz)).start()
            for zz in range(N_Z):
                if zz == me:
                    continue
                mk(zbu.at[zz], zbu.at[zz], zsend_u.at[zz], zrecv_u.at[zz],
                   (my_x, my_y, zz)).wait_recv()
                mk(zbl.at[zz], zbl.at[zz], zsend_l.at[zz], zrecv_l.at[zz],
                   (my_x, my_y, zz)).wait_recv()

    zsum_u = zstu[...]
    zsum_l = zstl[...]
    for zz in range(N_Z):
        zsum_u = zsum_u + jnp.where(my_z != zz, zbu[zz], 0.0)
        zsum_l = zsum_l + jnp.where(my_z != zz, zbl[zz], 0.0)
    zstu[...] = zsum_u
    zstl[...] = zsum_l

    x_did = (1 - my_x, my_y, my_z)
    y_did = (my_x, 1 - my_y, my_z)
    mk(zstu, xbu, xsu, xru, x_did).start()
    mk(zstl, xbl, xsl, xrl, x_did).start()
    mk(zstu, ybu.at[0], ysu.at[0], yru.at[0], y_did).start()
    mk(zstl, ybl.at[0], ysl.at[0], yrl.at[0], y_did).start()
    mk(xbu, xbu, xsu, xru, x_did).wait_recv()
    mk(xbl, xbl, xsl, xrl, x_did).wait_recv()
    mk(xbu, ybu.at[1], ysu.at[1], yru.at[1], y_did).start()
    mk(xbl, ybl.at[1], ysl.at[1], yrl.at[1], y_did).start()
    for j in range(2):
        mk(ybu.at[j], ybu.at[j], ysu.at[j], yru.at[j], y_did).wait_recv()
        mk(ybl.at[j], ybl.at[j], ysl.at[j], yrl.at[j], y_did).wait_recv()

    q_x = 2 * (1 - my_x) + my_y
    q_y0 = 2 * my_x + (1 - my_y)
    q_y1 = 2 * (1 - my_x) + (1 - my_y)
    for c in range(N_Z):
        o_ref[c:c + 1] = (
            jnp.where(q_idx == c, zstu[...], 0.0)
            + jnp.where(q_x == c, xbu[...], 0.0)
            + jnp.where(q_y0 == c, ybu[0], 0.0)
            + jnp.where(q_y1 == c, ybu[1], 0.0)
        )
        accl_ref[c:c + 1] = (
            jnp.where(q_idx == c, zstl[...], 0.0)
            + jnp.where(q_x == c, xbl[...], 0.0)
            + jnp.where(q_y0 == c, ybl[0], 0.0)
            + jnp.where(q_y1 == c, ybl[1], 0.0)
        )

    for me in range(N_Z):
        @pl.when(my_z == me)
        def _(me=me):
            for zz in range(N_Z):
                if zz == me:
                    continue
                mk(zstu, zbu.at[me], zsend_u.at[zz], zrecv_u.at[me],
                   (my_x, my_y, zz)).wait_send()
                mk(zstl, zbl.at[me], zsend_l.at[zz], zrecv_l.at[me],
                   (my_x, my_y, zz)).wait_send()
    mk(zstu, xbu, xsu, xru, x_did).wait_send()
    mk(zstl, xbl, xsl, xrl, x_did).wait_send()
    mk(zstu, ybu.at[0], ysu.at[0], yru.at[0], y_did).wait_send()
    mk(zstl, ybl.at[0], ysl.at[0], yrl.at[0], y_did).wait_send()
    mk(xbu, ybu.at[1], ysu.at[1], yru.at[1], y_did).wait_send()
    mk(xbl, ybl.at[1], ysl.at[1], yrl.at[1], y_did).wait_send()

    for bb in range(o_ref.shape[0]):
        for hh in range(H):
            lcol = accl_ref[bb, :, hh:hh + 1]
            o_ref[bb, :, hh, :] = o_ref[bb, :, hh, :] / lcol


def partial_only(Q, K, V):
    b, sq, h, d = Q.shape

    return pl.pallas_call(
        _partial_body,
        in_specs=[
            pl.BlockSpec(memory_space=pltpu.VMEM),
            pl.BlockSpec(memory_space=pltpu.MemorySpace.HBM),
            pl.BlockSpec(memory_space=pltpu.MemorySpace.HBM),
        ],
        out_specs=[
            pl.BlockSpec(memory_space=pltpu.VMEM),
            pl.BlockSpec(memory_space=pltpu.VMEM),
        ],
        out_shape=[
            jax.ShapeDtypeStruct((1, sq, h, d), jnp.float32),
            jax.ShapeDtypeStruct((1, sq, d), jnp.float32),
        ],
        scratch_shapes=[
            pltpu.VMEM((8, KV_CHUNK, d), jnp.float32),
            pltpu.VMEM((8, KV_CHUNK, d), jnp.float32),
            pltpu.SemaphoreType.DMA((8,)),
            pltpu.SemaphoreType.DMA((8,)),
        ],
    )(Q, K, V)


def allreduce_only(u, l):
    _, sq, h, d = u.shape
    b = N_Z

    return pl.pallas_call(
        _allreduce_body,
        in_specs=[
            pl.BlockSpec(memory_space=pltpu.VMEM),
            pl.BlockSpec(memory_space=pltpu.VMEM),
        ],
        out_specs=pl.BlockSpec(memory_space=pltpu.VMEM),
        out_shape=jax.ShapeDtypeStruct((b, sq, h, d), jnp.float32),
        scratch_shapes=[
            pltpu.VMEM((b, sq, d), jnp.float32),
            pltpu.VMEM((N_Z, 1, sq, h, d), jnp.float32),
            pltpu.VMEM((N_Z, 1, sq, d), jnp.float32),
            pltpu.VMEM((1, sq, h, d), jnp.float32),
            pltpu.VMEM((1, sq, d), jnp.float32),
            pltpu.VMEM((1, sq, h, d), jnp.float32),
            pltpu.VMEM((1, sq, d), jnp.float32),
            pltpu.VMEM((2, 1, sq, h, d), jnp.float32),
            pltpu.VMEM((2, 1, sq, d), jnp.float32),
            pltpu.SemaphoreType.DMA((N_Z,)),
            pltpu.SemaphoreType.DMA((N_Z,)),
            pltpu.SemaphoreType.DMA((N_Z,)),
            pltpu.SemaphoreType.DMA((N_Z,)),
            pltpu.SemaphoreType.DMA,
            pltpu.SemaphoreType.DMA,
            pltpu.SemaphoreType.DMA,
            pltpu.SemaphoreType.DMA,
            pltpu.SemaphoreType.DMA((2,)),
            pltpu.SemaphoreType.DMA((2,)),
            pltpu.SemaphoreType.DMA((2,)),
            pltpu.SemaphoreType.DMA((2,)),
        ],
        compiler_params=pltpu.CompilerParams(collective_id=0),
    )(u, l)


def kernel(Q, K, V):
    u, l = partial_only(Q, K, V)
    return allreduce_only(u, l)


# device time: 33195 ns/iter; 3.9400x vs baseline; 1.0121x over previous
import jax
import jax.numpy as jnp
from jax import lax
from jax.experimental import pallas as pl
from jax.experimental.pallas import tpu as pltpu

N_Z = 4
H = 8
D = 128
SCALE = D ** -0.5
KV_CHUNK = 2048


def _partial_body(q_ref, k_hbm, v_hbm, u_ref, l_ref, kbuf, vbuf, ksems, vsems):
    skv = k_hbm.shape[1]
    nc = skv // KV_CHUNK
    nit = H * nc
    q_idx = 2 * lax.axis_index("x") + lax.axis_index("y")

    l_ref[...] = jnp.zeros_like(l_ref)

    for qq in range(N_Z):
        @pl.when(q_idx == qq)
        def _(qq=qq):
            def params(i):
                return i // nc, i % nc

            def copies(i, slot):
                hh, cc = params(i)
                ck = pltpu.make_async_copy(
                    k_hbm.at[qq, pl.ds(cc * KV_CHUNK, KV_CHUNK), hh, :],
                    kbuf.at[slot],
                    ksems.at[slot],
                )
                cv = pltpu.make_async_copy(
                    v_hbm.at[qq, pl.ds(cc * KV_CHUNK, KV_CHUNK), hh, :],
                    vbuf.at[slot],
                    vsems.at[slot],
                )
                return ck, cv

            NBUF = kbuf.shape[0]
            for j in range(min(NBUF - 1, nit)):
                ckj, cvj = copies(j, j % NBUF)
                ckj.start()
                cvj.start()
            u_acc = None
            l_acc = None
            for i in range(nit):
                slot = i % NBUF
                if i + NBUF - 1 < nit:
                    ck2, cv2 = copies(i + NBUF - 1, (i + NBUF - 1) % NBUF)
                    ck2.start()
                    cv2.start()
                ck, cv = copies(i, slot)
                ck.wait()
                cv.wait()
                hh, cc = params(i)
                q = q_ref[qq, :, hh, :]
                k = kbuf[slot]
                v = vbuf[slot]
                s = lax.dot_general(
                    q, k, (((1,), (1,)), ((), ())),
                    preferred_element_type=jnp.float32,
                )
                p = jnp.exp(s * SCALE)
                lp = jnp.sum(p, axis=1, keepdims=True)
                up = lax.dot_general(
                    p, v, (((1,), (0,)), ((), ())),
                    preferred_element_type=jnp.float32,
                )
                if cc == 0:
                    u_acc, l_acc = up, lp
                else:
                    u_acc, l_acc = u_acc + up, l_acc + lp
                if cc == nc - 1:
                    u_ref[0, :, hh, :] = u_acc
                    l_ref[0, :, hh:hh + 1] = l_acc


def _allreduce_body(
    u_ref, l_ref, o_ref, accl_ref,
    zbu, zbl, zstu, zstl, xbu, xbl, ybu, ybl,
    zsend_u, zrecv_u, zsend_l, zrecv_l,
    xsu, xru, xsl, xrl, ysu, yru, ysl, yrl,
):
    my_x = lax.axis_index("x")
    my_y = lax.axis_index("y")
    my_z = lax.axis_index("z")
    q_idx = 2 * my_x + my_y

    def mk(src, dst, ssem, rsem, did):
        return pltpu.make_async_remote_copy(
            src_ref=src,
            dst_ref=dst,
            send_sem=ssem,
            recv_sem=rsem,
            device_id=did,
            device_id_type=pl.DeviceIdType.MESH,
        )

    barrier_sem = pltpu.get_barrier_semaphore()
    for zz in range(N_Z):
        @pl.when(my_z != zz)
        def _(zz=zz):
            pl.semaphore_signal(
                barrier_sem, inc=1,
                device_id=(my_x, my_y, zz),
                device_id_type=pl.DeviceIdType.MESH,
            )
    for did in ((1 - my_x, my_y, my_z), (my_x, 1 - my_y, my_z)):
        pl.semaphore_signal(
            barrier_sem, inc=1, device_id=did,
            device_id_type=pl.DeviceIdType.MESH,
        )
    pl.semaphore_wait(barrier_sem, N_Z - 1 + 2)

    zstu[...] = u_ref[...]
    zstl[...] = l_ref[...]

    for me in range(N_Z):
        @pl.when(my_z == me)
        def _(me=me):
            for zz in range(N_Z):
                if zz == me:
                    continue
                mk(zstu, zbu.at[me], zsend_u.at[zz], zrecv_u.at[me],
                   (my_x, my_y, zz)).start()
                mk(zstl, zbl.at[me], zsend_l.at[zz], zrecv_l.at[me],
                   (my_x, my_y, zz)).start()
            for zz in range(N_Z):
                if zz == me:
                    continue
                mk(zbu.at[zz], zbu.at[zz], zsend_u.at[zz], zrecv_u.at[zz],
                   (my_x, my_y, zz)).wait_recv()
                mk(zbl.at[zz], zbl.at[zz], zsend_l.at[zz], zrecv_l.at[zz],
                   (my_x, my_y, zz)).wait_recv()

    zsum_u = zstu[...]
    zsum_l = zstl[...]
    for zz in range(N_Z):
        zsum_u = zsum_u + jnp.where(my_z != zz, zbu[zz], 0.0)
        zsum_l = zsum_l + jnp.where(my_z != zz, zbl[zz], 0.0)
    zstu[...] = zsum_u
    zstl[...] = zsum_l

    x_did = (1 - my_x, my_y, my_z)
    y_did = (my_x, 1 - my_y, my_z)
    mk(zstu, xbu, xsu, xru, x_did).start()
    mk(zstl, xbl, xsl, xrl, x_did).start()
    mk(zstu, ybu.at[0], ysu.at[0], yru.at[0], y_did).start()
    mk(zstl, ybl.at[0], ysl.at[0], yrl.at[0], y_did).start()
    mk(xbu, xbu, xsu, xru, x_did).wait_recv()
    mk(xbl, xbl, xsl, xrl, x_did).wait_recv()
    mk(xbu, ybu.at[1], ysu.at[1], yru.at[1], y_did).start()
    mk(xbl, ybl.at[1], ysl.at[1], yrl.at[1], y_did).start()
    for j in range(2):
        mk(ybu.at[j], ybu.at[j], ysu.at[j], yru.at[j], y_did).wait_recv()
        mk(ybl.at[j], ybl.at[j], ysl.at[j], yrl.at[j], y_did).wait_recv()

    q_x = 2 * (1 - my_x) + my_y
    q_y0 = 2 * my_x + (1 - my_y)
    q_y1 = 2 * (1 - my_x) + (1 - my_y)
    for c in range(N_Z):
        o_ref[c:c + 1] = (
            jnp.where(q_idx == c, zstu[...], 0.0)
            + jnp.where(q_x == c, xbu[...], 0.0)
            + jnp.where(q_y0 == c, ybu[0], 0.0)
            + jnp.where(q_y1 == c, ybu[1], 0.0)
        )
        accl_ref[c:c + 1] = (
            jnp.where(q_idx == c, zstl[...], 0.0)
            + jnp.where(q_x == c, xbl[...], 0.0)
            + jnp.where(q_y0 == c, ybl[0], 0.0)
            + jnp.where(q_y1 == c, ybl[1], 0.0)
        )

    for me in range(N_Z):
        @pl.when(my_z == me)
        def _(me=me):
            for zz in range(N_Z):
                if zz == me:
                    continue
                mk(zstu, zbu.at[me], zsend_u.at[zz], zrecv_u.at[me],
                   (my_x, my_y, zz)).wait_send()
                mk(zstl, zbl.at[me], zsend_l.at[zz], zrecv_l.at[me],
                   (my_x, my_y, zz)).wait_send()
    mk(zstu, xbu, xsu, xru, x_did).wait_send()
    mk(zstl, xbl, xsl, xrl, x_did).wait_send()
    mk(zstu, ybu.at[0], ysu.at[0], yru.at[0], y_did).wait_send()
    mk(zstl, ybl.at[0], ysl.at[0], yrl.at[0], y_did).wait_send()
    mk(xbu, ybu.at[1], ysu.at[1], yru.at[1], y_did).wait_send()
    mk(xbl, ybl.at[1], ysl.at[1], yrl.at[1], y_did).wait_send()

    for bb in range(o_ref.shape[0]):
        for hh in range(H):
            lcol = accl_ref[bb, :, hh:hh + 1]
            o_ref[bb, :, hh, :] = o_ref[bb, :, hh, :] / lcol


def partial_only(Q, K, V):
    b, sq, h, d = Q.shape

    return pl.pallas_call(
        _partial_body,
        in_specs=[
            pl.BlockSpec(memory_space=pltpu.VMEM),
            pl.BlockSpec(memory_space=pltpu.MemorySpace.HBM),
            pl.BlockSpec(memory_space=pltpu.MemorySpace.HBM),
        ],
        out_specs=[
            pl.BlockSpec(memory_space=pltpu.VMEM),
            pl.BlockSpec(memory_space=pltpu.VMEM),
        ],
        out_shape=[
            jax.ShapeDtypeStruct((1, sq, h, d), jnp.float32),
            jax.ShapeDtypeStruct((1, sq, d), jnp.float32),
        ],
        scratch_shapes=[
            pltpu.VMEM((8, KV_CHUNK, d), jnp.float32),
            pltpu.VMEM((8, KV_CHUNK, d), jnp.float32),
            pltpu.SemaphoreType.DMA((8,)),
            pltpu.SemaphoreType.DMA((8,)),
        ],
    )(Q, K, V)


def allreduce_only(u, l):
    _, sq, h, d = u.shape
    b = N_Z

    return pl.pallas_call(
        _allreduce_body,
        in_specs=[
            pl.BlockSpec(memory_space=pltpu.VMEM),
            pl.BlockSpec(memory_space=pltpu.VMEM),
        ],
        out_specs=pl.BlockSpec(memory_space=pltpu.VMEM),
        out_shape=jax.ShapeDtypeStruct((b, sq, h, d), jnp.float32),
        scratch_shapes=[
            pltpu.VMEM((b, sq, d), jnp.float32),
            pltpu.VMEM((N_Z, 1, sq, h, d), jnp.float32),
            pltpu.VMEM((N_Z, 1, sq, d), jnp.float32),
            pltpu.VMEM((1, sq, h, d), jnp.float32),
            pltpu.VMEM((1, sq, d), jnp.float32),
            pltpu.VMEM((1, sq, h, d), jnp.float32),
            pltpu.VMEM((1, sq, d), jnp.float32),
            pltpu.VMEM((2, 1, sq, h, d), jnp.float32),
            pltpu.VMEM((2, 1, sq, d), jnp.float32),
            pltpu.SemaphoreType.DMA((N_Z,)),
            pltpu.SemaphoreType.DMA((N_Z,)),
            pltpu.SemaphoreType.DMA((N_Z,)),
            pltpu.SemaphoreType.DMA((N_Z,)),
            pltpu.SemaphoreType.DMA,
            pltpu.SemaphoreType.DMA,
            pltpu.SemaphoreType.DMA,
            pltpu.SemaphoreType.DMA,
            pltpu.SemaphoreType.DMA((2,)),
            pltpu.SemaphoreType.DMA((2,)),
            pltpu.SemaphoreType.DMA((2,)),
            pltpu.SemaphoreType.DMA((2,)),
        ],
        compiler_params=pltpu.CompilerParams(collective_id=0),
    )(u, l)


def kernel(Q, K, V):
    u, l = partial_only(Q, K, V)
    return allreduce_only(u, l)
